# baseline (speedup 1.0000x reference)
import jax
import jax.numpy as jnp
from jax.experimental import pallas as pl
from jax.experimental.pallas import tpu as pltpu

_NUM_CLASSES, _NUM_MAPS = 8, 4
_NC = _NUM_CLASSES * _NUM_MAPS


def _wscnet_kernel(f_ref, w_ref, cb_ref, fcb_ref, xv_ref, res_ref):
    f = f_ref[0]
    w = w_ref[...]
    hw = f.shape[1]

    y = jnp.dot(w, f, preferred_element_type=jnp.float32)
    x = y[:_NC, :] + cb_ref[...]
    u1 = y[_NC:_NC + _NUM_CLASSES, :]
    u2 = y[_NC + _NUM_CLASSES:, :]

    gmp = jnp.max(x, axis=1, keepdims=True)

    sel = (jax.lax.broadcasted_iota(jnp.int32, (_NC, _NUM_CLASSES), 0)
           // _NUM_MAPS
           == jax.lax.broadcasted_iota(jnp.int32, (_NC, _NUM_CLASSES), 1))
    xv_row = jax.lax.dot_general(
        gmp, sel.astype(jnp.float32), (((0,), (0,)), ((), ())),
        preferred_element_type=jnp.float32) * (1.0 / _NUM_MAPS)

    q = (jax.lax.broadcasted_iota(jnp.int32, (_NC, _NC), 0) // _NUM_MAPS
         == jax.lax.broadcasted_iota(jnp.int32, (_NC, _NC), 1) // _NUM_MAPS)
    coeff = jnp.dot(q.astype(jnp.float32), gmp,
                    preferred_element_type=jnp.float32) * (1.0 / 128.0)

    s = jax.lax.dot_general(coeff, x, (((0,), (0,)), ((), ())),
                            preferred_element_type=jnp.float32)

    v = u1 + u2 * s
    ones = jnp.ones((hw, 1), jnp.float32)
    res_row = jax.lax.dot_general(
        ones, v, (((0,), (1,)), ((), ())),
        preferred_element_type=jnp.float32) * (1.0 / hw) + fcb_ref[...]

    xv_ref[0] = xv_row
    res_ref[0] = res_row


def kernel(features, conv_w, conv_b, fc_w, fc_b):
    B, C, H, W = features.shape
    HW = H * W
    f3 = features.reshape(B, C, HW)
    w_stack = jnp.concatenate([conv_w, fc_w[:, :C], fc_w[:, C:]], axis=0)
    cb2 = conv_b.reshape(_NC, 1)
    fcb2 = fc_b.reshape(1, _NUM_CLASSES)

    out_sds = jax.ShapeDtypeStruct((B, 1, _NUM_CLASSES), jnp.float32)
    xv, res = pl.pallas_call(
        _wscnet_kernel,
        grid=(B,),
        in_specs=[
            pl.BlockSpec((1, C, HW), lambda b: (b, 0, 0)),
            pl.BlockSpec((_NC + 2 * _NUM_CLASSES, C), lambda b: (0, 0)),
            pl.BlockSpec((_NC, 1), lambda b: (0, 0)),
            pl.BlockSpec((1, _NUM_CLASSES), lambda b: (0, 0)),
        ],
        out_specs=[
            pl.BlockSpec((1, 1, _NUM_CLASSES), lambda b: (b, 0, 0)),
            pl.BlockSpec((1, 1, _NUM_CLASSES), lambda b: (b, 0, 0)),
        ],
        out_shape=[out_sds, out_sds],
        compiler_params=pltpu.CompilerParams(
            dimension_semantics=("parallel",)),
    )(f3, w_stack, cb2, fcb2)
    return (xv.reshape(B, _NUM_CLASSES), res.reshape(B, _NUM_CLASSES))


# speedup vs baseline: 1.1619x; 1.1619x over previous
import jax
import jax.numpy as jnp
from jax.experimental import pallas as pl
from jax.experimental.pallas import tpu as pltpu

_NUM_CLASSES, _NUM_MAPS = 8, 4
_NC = _NUM_CLASSES * _NUM_MAPS
_BB = 8


def _one_batch(f, w, cb, fcb):
    hw = f.shape[1]
    y = jnp.dot(w, f, preferred_element_type=jnp.float32)
    x = y[:_NC, :] + cb
    u1 = y[_NC:_NC + _NUM_CLASSES, :]
    u2 = y[_NC + _NUM_CLASSES:, :]

    gmp = jnp.max(x, axis=1, keepdims=True)

    sel = (jax.lax.broadcasted_iota(jnp.int32, (_NC, _NUM_CLASSES), 0)
           // _NUM_MAPS
           == jax.lax.broadcasted_iota(jnp.int32, (_NC, _NUM_CLASSES), 1))
    xv_row = jax.lax.dot_general(
        gmp, sel.astype(jnp.float32), (((0,), (0,)), ((), ())),
        preferred_element_type=jnp.float32) * (1.0 / _NUM_MAPS)

    q = (jax.lax.broadcasted_iota(jnp.int32, (_NC, _NC), 0) // _NUM_MAPS
         == jax.lax.broadcasted_iota(jnp.int32, (_NC, _NC), 1) // _NUM_MAPS)
    coeff = jnp.dot(q.astype(jnp.float32), gmp,
                    preferred_element_type=jnp.float32) * (1.0 / 128.0)

    s = jax.lax.dot_general(coeff, x, (((0,), (0,)), ((), ())),
                            preferred_element_type=jnp.float32)

    v = u1 + u2 * s
    ones = jnp.ones((hw, 1), jnp.float32)
    res_row = jax.lax.dot_general(
        ones, v, (((0,), (1,)), ((), ())),
        preferred_element_type=jnp.float32) * (1.0 / hw) + fcb
    return xv_row, res_row


def _wscnet_kernel(f_ref, w_ref, cb_ref, fcb_ref, xv_ref, res_ref):
    w = w_ref[...]
    cb = cb_ref[...]
    fcb = fcb_ref[...]
    for k in range(_BB):
        xv_row, res_row = _one_batch(f_ref[k], w, cb, fcb)
        xv_ref[k] = xv_row
        res_ref[k] = res_row


def kernel(features, conv_w, conv_b, fc_w, fc_b):
    B, C, H, W = features.shape
    HW = H * W
    f3 = features.reshape(B, C, HW)
    w_stack = jnp.concatenate([conv_w, fc_w[:, :C], fc_w[:, C:]], axis=0)
    cb2 = conv_b.reshape(_NC, 1)
    fcb2 = fc_b.reshape(1, _NUM_CLASSES)

    out_sds = jax.ShapeDtypeStruct((B, 1, _NUM_CLASSES), jnp.float32)
    xv, res = pl.pallas_call(
        _wscnet_kernel,
        grid=(B // _BB,),
        in_specs=[
            pl.BlockSpec((_BB, C, HW), lambda b: (b, 0, 0)),
            pl.BlockSpec((_NC + 2 * _NUM_CLASSES, C), lambda b: (0, 0)),
            pl.BlockSpec((_NC, 1), lambda b: (0, 0)),
            pl.BlockSpec((1, _NUM_CLASSES), lambda b: (0, 0)),
        ],
        out_specs=[
            pl.BlockSpec((_BB, 1, _NUM_CLASSES), lambda b: (b, 0, 0)),
            pl.BlockSpec((_BB, 1, _NUM_CLASSES), lambda b: (b, 0, 0)),
        ],
        out_shape=[out_sds, out_sds],
        compiler_params=pltpu.CompilerParams(
            dimension_semantics=("parallel",)),
    )(f3, w_stack, cb2, fcb2)
    return (xv.reshape(B, _NUM_CLASSES), res.reshape(B, _NUM_CLASSES))


# speedup vs baseline: 4.6947x; 4.0407x over previous
import jax
import jax.numpy as jnp
from jax.experimental import pallas as pl
from jax.experimental.pallas import tpu as pltpu

_NUM_CLASSES, _NUM_MAPS = 8, 4
_NC = _NUM_CLASSES * _NUM_MAPS
_NW = _NC + 2 * _NUM_CLASSES
_NP = 7
_P = 28


def _iota2(shape, d0, d1, fn):
    a = jax.lax.broadcasted_iota(jnp.int32, shape, d0)
    b = jax.lax.broadcasted_iota(jnp.int32, shape, d1)
    return fn(a, b).astype(jnp.float32)


def _wscnet_kernel(f_ref, wt_ref, cb_ref, fcb_ref, xv_ref, res_ref, y_scr):
    p = pl.program_id(1)
    bb = f_ref.shape[1]
    f2 = f_ref[...].reshape(_P * bb, 2048)
    y_scr[p] = jnp.dot(f2, wt_ref[...], preferred_element_type=jnp.float32)

    @pl.when(p == _NP - 1)
    def _finale():
        hw = _NP * _P
        y3 = y_scr[...].reshape(hw, bb, _NW)
        cb = cb_ref[...]

        gmp = jnp.max(y3, axis=0) + cb

        sel = _iota2((_NW, _NUM_CLASSES), 0, 1,
                     lambda o, k: (o // _NUM_MAPS) == k)
        xv = jnp.dot(gmp, sel, preferred_element_type=jnp.float32) * 0.25

        selt = _iota2((_NUM_CLASSES, _NW), 1, 0,
                      lambda o, k: (o // _NUM_MAPS) == k)
        coeff = jnp.dot(xv, selt,
                        preferred_element_type=jnp.float32) * (1.0 / 32.0)

        beta = jnp.sum(coeff * cb, axis=1, keepdims=True)[None]
        s3 = jnp.sum(y3 * coeff[None], axis=2, keepdims=True) + beta

        r_u1 = jnp.sum(y3, axis=0)
        r_u2 = jnp.sum(y3 * s3, axis=0)
        su1 = _iota2((_NW, _NUM_CLASSES), 0, 1, lambda o, i: o == _NC + i)
        su2 = _iota2((_NW, _NUM_CLASSES), 0, 1,
                     lambda o, i: o == _NC + _NUM_CLASSES + i)
        res = (jnp.dot(r_u1, su1, preferred_element_type=jnp.float32)
               + jnp.dot(r_u2, su2, preferred_element_type=jnp.float32)
               ) * (1.0 / hw) + fcb_ref[...]

        xv_ref[...] = xv
        res_ref[...] = res


def kernel(features, conv_w, conv_b, fc_w, fc_b):
    B, C, H, W = features.shape
    HW = H * W
    fp = features.transpose(2, 3, 0, 1).reshape(HW, B, C)
    wt = jnp.concatenate([conv_w, fc_w[:, :C], fc_w[:, C:]], axis=0).T
    cb_pad = jnp.concatenate(
        [conv_b, jnp.zeros((_NW - _NC,), jnp.float32)]).reshape(1, _NW)
    fcb2 = fc_b.reshape(1, _NUM_CLASSES)

    bb = B // 2
    out_sds = jax.ShapeDtypeStruct((B, _NUM_CLASSES), jnp.float32)
    xv, res = pl.pallas_call(
        _wscnet_kernel,
        grid=(2, _NP),
        in_specs=[
            pl.BlockSpec((_P, bb, C), lambda i, p: (p, i, 0)),
            pl.BlockSpec((C, _NW), lambda i, p: (0, 0)),
            pl.BlockSpec((1, _NW), lambda i, p: (0, 0)),
            pl.BlockSpec((1, _NUM_CLASSES), lambda i, p: (0, 0)),
        ],
        out_specs=[
            pl.BlockSpec((bb, _NUM_CLASSES), lambda i, p: (i, 0)),
            pl.BlockSpec((bb, _NUM_CLASSES), lambda i, p: (i, 0)),
        ],
        out_shape=[out_sds, out_sds],
        scratch_shapes=[pltpu.VMEM((_NP, _P * bb, _NW), jnp.float32)],
        compiler_params=pltpu.CompilerParams(
            dimension_semantics=("parallel", "arbitrary")),
    )(fp, wt, cb_pad, fcb2)
    return (xv, res)
